# baseline (speedup 1.0000x reference)
import jax
import jax.numpy as jnp
from jax.experimental import pallas as pl

_POOL = 4
_J_IN = 6
_J_OUT = 24


def _unpool_body(in_ref, out_ref):
    x = in_ref[0]
    F = x.shape[0]
    D = x.shape[1] // _J_IN
    z = jnp.zeros((F, (_POOL - 1) * D), x.dtype)
    pieces = []
    for k in range(_J_IN):
        pieces.append(x[:, k * D:(k + 1) * D])
        pieces.append(z)
    y = jnp.concatenate(pieces, axis=1)
    out_ref[0] = jnp.concatenate([y] * _POOL, axis=1)


def kernel(latent):
    B, F, J, D = latent.shape
    row = _POOL * _J_OUT * D
    out3 = pl.pallas_call(
        _unpool_body,
        grid=(B,),
        in_specs=[pl.BlockSpec((1, F, J * D), lambda b: (b, 0, 0))],
        out_specs=pl.BlockSpec((1, F, row), lambda b: (b, 0, 0)),
        out_shape=jax.ShapeDtypeStruct((B, F, row), latent.dtype),
    )(latent.reshape(B, F, J * D))
    return out3.reshape(B, F * _POOL, _J_OUT, D)


# speedup vs baseline: 2.2382x; 2.2382x over previous
import jax
import jax.numpy as jnp
from jax.experimental import pallas as pl

_POOL = 4
_J_IN = 6
_J_OUT = 24
_FSPLIT = 2


def _unpool_body(in_ref, out_ref):
    x = in_ref[0]
    F, J, D = x.shape
    xb = jnp.broadcast_to(x[:, None, :, None, :], (F, _POOL, J, _POOL, D))
    r = jax.lax.broadcasted_iota(jnp.int32, (F, _POOL, J, _POOL, D), 3)
    out_ref[0] = jnp.where(r == 0, xb, 0.0)


def kernel(latent):
    B, F, J, D = latent.shape
    Fb = F // _FSPLIT
    out6 = pl.pallas_call(
        _unpool_body,
        grid=(B, _FSPLIT),
        in_specs=[pl.BlockSpec((1, Fb, J, D), lambda b, f: (b, f, 0, 0))],
        out_specs=pl.BlockSpec(
            (1, Fb, _POOL, J, _POOL, D), lambda b, f: (b, f, 0, 0, 0, 0)
        ),
        out_shape=jax.ShapeDtypeStruct((B, F, _POOL, J, _POOL, D), latent.dtype),
    )(latent)
    return out6.reshape(B, F * _POOL, _J_OUT, D)


# speedup vs baseline: 3.3819x; 1.5110x over previous
import jax
import jax.numpy as jnp
from jax.experimental import pallas as pl

_POOL = 4
_J_IN = 6
_J_OUT = 24
_BBLK = 2


def _unpool_body(in_ref, out_ref):
    x = in_ref[...]
    Bb, F, J, D = x.shape
    xb = jnp.broadcast_to(
        x[:, :, None, :, None, :], (Bb, F, _POOL, J, _POOL, D)
    )
    r = jax.lax.broadcasted_iota(jnp.int32, (Bb, F, _POOL, J, _POOL, D), 4)
    out_ref[...] = jnp.where(r == 0, xb, 0.0)


def kernel(latent):
    B, F, J, D = latent.shape
    out6 = pl.pallas_call(
        _unpool_body,
        grid=(B // _BBLK,),
        in_specs=[pl.BlockSpec((_BBLK, F, J, D), lambda b: (b, 0, 0, 0))],
        out_specs=pl.BlockSpec(
            (_BBLK, F, _POOL, J, _POOL, D), lambda b: (b, 0, 0, 0, 0, 0)
        ),
        out_shape=jax.ShapeDtypeStruct((B, F, _POOL, J, _POOL, D), latent.dtype),
    )(latent)
    return out6.reshape(B, F * _POOL, _J_OUT, D)


# speedup vs baseline: 3.4141x; 1.0095x over previous
import jax
import jax.numpy as jnp
from jax.experimental import pallas as pl

_POOL = 4
_J_IN = 6
_J_OUT = 24
_BBLK = 4


def _unpool_body(in_ref, out_ref):
    x = in_ref[...]
    Bb, F, J, D = x.shape
    xb = jnp.broadcast_to(
        x[:, :, None, :, None, :], (Bb, F, _POOL, J, _POOL, D)
    )
    r = jax.lax.broadcasted_iota(jnp.int32, (Bb, F, _POOL, J, _POOL, D), 4)
    out_ref[...] = jnp.where(r == 0, xb, 0.0)


def kernel(latent):
    B, F, J, D = latent.shape
    out6 = pl.pallas_call(
        _unpool_body,
        grid=(B // _BBLK,),
        in_specs=[pl.BlockSpec((_BBLK, F, J, D), lambda b: (b, 0, 0, 0))],
        out_specs=pl.BlockSpec(
            (_BBLK, F, _POOL, J, _POOL, D), lambda b: (b, 0, 0, 0, 0, 0)
        ),
        out_shape=jax.ShapeDtypeStruct((B, F, _POOL, J, _POOL, D), latent.dtype),
    )(latent)
    return out6.reshape(B, F * _POOL, _J_OUT, D)
